# Initial kernel scaffold; baseline (speedup 1.0000x reference)
import jax
import jax.numpy as jnp
from jax.experimental import pallas as pl


def kernel(cue, patches):
    raise NotImplementedError("write your pallas kernel here")


# speedup vs baseline: 3.8958x; 3.8958x over previous
import jax
import jax.numpy as jnp
from jax.experimental import pallas as pl

_H = 32
_W = 32
_R = 1


def _buddy_kernel(cue_ref, patches_ref, out_ref):
    patches = patches_ref[0]
    cue = cue_ref[0]
    sim = jax.lax.dot_general(
        cue, patches, (((1,), (1,)), ((), ())),
        preferred_element_type=jnp.float32)
    idx = jnp.argmax(sim, axis=1)
    h = idx // _W
    w = idx % _W
    pos = jax.lax.broadcasted_iota(jnp.int32, sim.shape, 1)
    ph = pos // _W
    pw = pos % _W
    mask = ((jnp.abs(ph - h[:, None]) <= _R) &
            (jnp.abs(pw - w[:, None]) <= _R)).astype(jnp.float32)
    cnt = mask.sum(axis=1, keepdims=True)
    roi = jax.lax.dot_general(
        mask, patches, (((1,), (0,)), ((), ())),
        preferred_element_type=jnp.float32) / cnt
    out_ref[0] = roi


def kernel(cue, patches):
    B, K, D = cue.shape
    _, H, W, _ = patches.shape
    patches_flat = patches.reshape(B, H * W, D)
    return pl.pallas_call(
        _buddy_kernel,
        grid=(B,),
        in_specs=[
            pl.BlockSpec((1, K, D), lambda b: (b, 0, 0)),
            pl.BlockSpec((1, H * W, D), lambda b: (b, 0, 0)),
        ],
        out_specs=pl.BlockSpec((1, K, D), lambda b: (b, 0, 0)),
        out_shape=jax.ShapeDtypeStruct((B, K, D), jnp.float32),
    )(cue, patches_flat)
